# baseline (speedup 1.0000x reference)
import functools

import jax
import jax.numpy as jnp
from jax import lax
from jax.experimental import pallas as pl
from jax.experimental.pallas import tpu as pltpu
from jax.experimental.pallas import tpu_sc as plsc

_B0 = 32
_B1 = 64
_S0 = 8
_S1 = 8
_STEPS = _S0 + _S1


def _tc_body(theta0_ref, theta1_ref, map_ref, obs0_ref,
             proc1_ref, l0_ref, proc0_sc):
    i = pl.program_id(0)

    @pl.when(i < _S0)
    def _theta0():
        s0 = jnp.sum(jnp.exp(theta0_ref[...]), axis=1)
        proc0_sc[pl.ds(i * _B0, _B0), :] = s0

    @pl.when(i == _S0)
    def _loss0():
        p = jnp.dot(map_ref[...], proc0_sc[...],
                    preferred_element_type=jnp.float32)
        d0 = obs0_ref[...] - p
        l0 = jnp.sum(d0 * d0, keepdims=True) * (1.0 / (2048.0 * 256.0))
        l0_ref[...] = jnp.broadcast_to(l0.reshape(1, 1), (1, 128))

    @pl.when(i >= _S0)
    def _theta1():
        proc1_ref[...] = jnp.sum(jnp.exp(theta1_ref[...]), axis=2)


_SC_W = 16
_RPW = 512 // _SC_W
_ORW = 64 // _SC_W


def _sc_body(proc1_hbm, idx_hbm, obs1_hbm, l0_hbm, zeros_hbm, out_hbm,
             rows_v, idx_v, a_v, obs_v, pv_v, pall_v, l0_v, out_v,
             acc_sh, part_sh):
    w = lax.axis_index("s")

    pltpu.sync_copy(idx_hbm.at[pl.ds(w * _RPW, _RPW)], idx_v)
    pltpu.sync_copy(proc1_hbm.at[pl.ds(w * _RPW, _RPW)], rows_v)

    pltpu.sync_copy(zeros_hbm.at[pl.ds(w * _ORW, _ORW)],
                    acc_sh.at[pl.ds(w * _ORW, _ORW)])
    plsc.subcore_barrier()

    for rnd in range(_SC_W):
        @pl.when(w == rnd)
        def _scatter(rnd=rnd):
            pltpu.sync_copy(rows_v, acc_sh.at[idx_v], add=True)
        plsc.subcore_barrier()

    @pl.when(w == 0)
    def _finish():
        acc = jnp.zeros((16,), jnp.float32)
        for blk in range(64 // _ORW):
            pltpu.sync_copy(acc_sh.at[pl.ds(blk * _ORW, _ORW)], a_v)
            pltpu.sync_copy(obs1_hbm.at[pl.ds(blk * _ORW, _ORW)], obs_v)
            for r in range(_ORW):
                for l in range(128 // 16):
                    d = obs_v[r, pl.ds(l * 16, 16)] - a_v[r, pl.ds(l * 16, 16)]
                    acc = acc + d * d
        lanes = jnp.arange(16, dtype=jnp.int32)
        for s in (1, 2, 4, 8):
            pv_v[...] = acc
            acc = acc + plsc.load_gather(pv_v, [(lanes + s) & 15])
        pltpu.sync_copy(l0_hbm, l0_v)
        l0vec = l0_v[0, pl.ds(0, 16)]
        out_v[...] = 0.5 * (l0vec + acc * (1.0 / (64.0 * 128.0)))
        pltpu.sync_copy(out_v, out_hbm)


@functools.partial(
    pl.kernel,
    out_type=jax.ShapeDtypeStruct((16,), jnp.float32),
    mesh=plsc.VectorSubcoreMesh(core_axis_name="c", subcore_axis_name="s",
                                num_cores=1),
    compiler_params=pltpu.CompilerParams(needs_layout_passes=False),
    scratch_types=[
        pltpu.VMEM((_RPW, 128), jnp.float32),
        pltpu.VMEM((_RPW,), jnp.int32),
        pltpu.VMEM((_ORW, 128), jnp.float32),
        pltpu.VMEM((_ORW, 128), jnp.float32),
        pltpu.VMEM((16,), jnp.float32),
        pltpu.VMEM((_SC_W, 16), jnp.float32),
        pltpu.VMEM((1, 128), jnp.float32),
        pltpu.VMEM((16,), jnp.float32),
        pltpu.VMEM_SHARED((64, 128), jnp.float32),
        pltpu.VMEM_SHARED((_SC_W, 16), jnp.float32),
    ],
)
def _sc_finish(proc1_hbm, idx_hbm, obs1_hbm, l0_hbm, zeros_hbm, out_hbm,
               *scratch):
    _sc_body(proc1_hbm, idx_hbm, obs1_hbm, l0_hbm, zeros_hbm, out_hbm,
             *scratch)


def kernel(theta0, theta1, obs0, obs1, mapping0, idx1):
    theta0_t = jnp.transpose(theta0, (0, 2, 1))
    proc1, l0 = pl.pallas_call(
        _tc_body,
        grid=(_STEPS,),
        in_specs=[
            pl.BlockSpec((_B0, 64, 256), lambda i: (jnp.minimum(i, _S0 - 1), 0, 0)),
            pl.BlockSpec((_B1, 128, 128), lambda i: (jnp.maximum(i - _S0, 0), 0, 0)),
            pl.BlockSpec((2048, 256), lambda i: (0, 0)),
            pl.BlockSpec((2048, 256), lambda i: (0, 0)),
        ],
        out_specs=[
            pl.BlockSpec((_B1, 128), lambda i: (jnp.maximum(i - _S0, 0), 0)),
            pl.BlockSpec((1, 128), lambda i: (0, 0)),
        ],
        out_shape=[
            jax.ShapeDtypeStruct((512, 128), jnp.float32),
            jax.ShapeDtypeStruct((1, 128), jnp.float32),
        ],
        scratch_shapes=[
            pltpu.VMEM((256, 256), jnp.float32),
        ],
        compiler_params=pltpu.CompilerParams(
            dimension_semantics=("arbitrary",),
        ),
    )(theta0_t, theta1, mapping0, obs0)
    zeros64 = jnp.zeros((64, 128), jnp.float32)
    out = _sc_finish(proc1, idx1.astype(jnp.int32), obs1, l0, zeros64)
    return out[0]


# speedup vs baseline: 1.0673x; 1.0673x over previous
import functools

import jax
import jax.numpy as jnp
from jax import lax
from jax.experimental import pallas as pl
from jax.experimental.pallas import tpu as pltpu
from jax.experimental.pallas import tpu_sc as plsc

_B0 = 32
_B1 = 64
_S0 = 8
_S1 = 8
_STEPS = _S0 + _S1


def _tc_body(theta0_ref, theta1_ref, map_ref, obs0_ref,
             proc1_ref, l0_ref, proc0_sc):
    i = pl.program_id(0)

    @pl.when(i < _S0)
    def _theta0():
        s0 = jnp.sum(jnp.exp(theta0_ref[...]), axis=1)
        proc0_sc[pl.ds(i * _B0, _B0), :] = s0

    @pl.when(i == _S0)
    def _loss0():
        p = jnp.dot(map_ref[...], proc0_sc[...],
                    preferred_element_type=jnp.float32)
        d0 = obs0_ref[...] - p
        l0 = jnp.sum(d0 * d0, keepdims=True) * (1.0 / (2048.0 * 256.0))
        l0_ref[...] = jnp.broadcast_to(l0.reshape(1, 1), (1, 128))

    @pl.when(i >= _S0)
    def _theta1():
        proc1_ref[...] = jnp.sum(jnp.exp(theta1_ref[...]), axis=2)


_SC_W = 16
_RPW = 512 // _SC_W
_ORW = 64 // _SC_W


def _sc_body(proc1_hbm, idx_hbm, obs1_hbm, l0_hbm, zeros_hbm, out_hbm,
             rows_v, idx_v, a_v, obs_v, pv_v, pall_v, l0_v, out_v,
             acc_sh, part_sh):
    w = lax.axis_index("s")

    pltpu.sync_copy(idx_hbm.at[pl.ds(w * _RPW, _RPW)], idx_v)
    pltpu.sync_copy(proc1_hbm.at[pl.ds(w * _RPW, _RPW)], rows_v)

    pltpu.sync_copy(zeros_hbm.at[pl.ds(w * _ORW, _ORW)],
                    acc_sh.at[pl.ds(w * _ORW, _ORW)])
    plsc.subcore_barrier()

    pltpu.sync_copy(rows_v, acc_sh.at[idx_v], add=True)
    plsc.subcore_barrier()

    @pl.when(w == 0)
    def _finish():
        acc = jnp.zeros((16,), jnp.float32)
        for blk in range(64 // _ORW):
            pltpu.sync_copy(acc_sh.at[pl.ds(blk * _ORW, _ORW)], a_v)
            pltpu.sync_copy(obs1_hbm.at[pl.ds(blk * _ORW, _ORW)], obs_v)
            for r in range(_ORW):
                for l in range(128 // 16):
                    d = obs_v[r, pl.ds(l * 16, 16)] - a_v[r, pl.ds(l * 16, 16)]
                    acc = acc + d * d
        lanes = jnp.arange(16, dtype=jnp.int32)
        for s in (1, 2, 4, 8):
            pv_v[...] = acc
            acc = acc + plsc.load_gather(pv_v, [(lanes + s) & 15])
        pltpu.sync_copy(l0_hbm, l0_v)
        l0vec = l0_v[0, pl.ds(0, 16)]
        out_v[...] = 0.5 * (l0vec + acc * (1.0 / (64.0 * 128.0)))
        pltpu.sync_copy(out_v, out_hbm)


@functools.partial(
    pl.kernel,
    out_type=jax.ShapeDtypeStruct((16,), jnp.float32),
    mesh=plsc.VectorSubcoreMesh(core_axis_name="c", subcore_axis_name="s",
                                num_cores=1),
    compiler_params=pltpu.CompilerParams(needs_layout_passes=False),
    scratch_types=[
        pltpu.VMEM((_RPW, 128), jnp.float32),
        pltpu.VMEM((_RPW,), jnp.int32),
        pltpu.VMEM((_ORW, 128), jnp.float32),
        pltpu.VMEM((_ORW, 128), jnp.float32),
        pltpu.VMEM((16,), jnp.float32),
        pltpu.VMEM((_SC_W, 16), jnp.float32),
        pltpu.VMEM((1, 128), jnp.float32),
        pltpu.VMEM((16,), jnp.float32),
        pltpu.VMEM_SHARED((64, 128), jnp.float32),
        pltpu.VMEM_SHARED((_SC_W, 16), jnp.float32),
    ],
)
def _sc_finish(proc1_hbm, idx_hbm, obs1_hbm, l0_hbm, zeros_hbm, out_hbm,
               *scratch):
    _sc_body(proc1_hbm, idx_hbm, obs1_hbm, l0_hbm, zeros_hbm, out_hbm,
             *scratch)


def kernel(theta0, theta1, obs0, obs1, mapping0, idx1):
    theta0_t = jnp.transpose(theta0, (0, 2, 1))
    proc1, l0 = pl.pallas_call(
        _tc_body,
        grid=(_STEPS,),
        in_specs=[
            pl.BlockSpec((_B0, 64, 256), lambda i: (jnp.minimum(i, _S0 - 1), 0, 0)),
            pl.BlockSpec((_B1, 128, 128), lambda i: (jnp.maximum(i - _S0, 0), 0, 0)),
            pl.BlockSpec((2048, 256), lambda i: (0, 0)),
            pl.BlockSpec((2048, 256), lambda i: (0, 0)),
        ],
        out_specs=[
            pl.BlockSpec((_B1, 128), lambda i: (jnp.maximum(i - _S0, 0), 0)),
            pl.BlockSpec((1, 128), lambda i: (0, 0)),
        ],
        out_shape=[
            jax.ShapeDtypeStruct((512, 128), jnp.float32),
            jax.ShapeDtypeStruct((1, 128), jnp.float32),
        ],
        scratch_shapes=[
            pltpu.VMEM((256, 256), jnp.float32),
        ],
        compiler_params=pltpu.CompilerParams(
            dimension_semantics=("arbitrary",),
        ),
    )(theta0_t, theta1, mapping0, obs0)
    zeros64 = jnp.zeros((64, 128), jnp.float32)
    out = _sc_finish(proc1, idx1.astype(jnp.int32), obs1, l0, zeros64)
    return out[0]


# speedup vs baseline: 1.1229x; 1.0521x over previous
import functools

import jax
import jax.numpy as jnp
from jax import lax
from jax.experimental import pallas as pl
from jax.experimental.pallas import tpu as pltpu
from jax.experimental.pallas import tpu_sc as plsc

_B0 = 64
_B1 = 128
_S0 = 4
_S1 = 4
_STEPS = _S0 + _S1


def _tc_body(theta0_ref, theta1_ref, map_ref, obs0_ref,
             proc1_ref, l0_ref, proc0_sc):
    i = pl.program_id(0)

    @pl.when(i < _S0)
    def _theta0():
        s0 = jnp.sum(jnp.exp(theta0_ref[...]), axis=1)
        proc0_sc[pl.ds(i * _B0, _B0), :] = s0

    @pl.when(i == _S0)
    def _loss0():
        p = jnp.dot(map_ref[...], proc0_sc[...],
                    preferred_element_type=jnp.float32)
        d0 = obs0_ref[...] - p
        l0 = jnp.sum(d0 * d0, keepdims=True) * (1.0 / (2048.0 * 256.0))
        l0_ref[...] = jnp.broadcast_to(l0.reshape(1, 1), (1, 128))

    @pl.when(i >= _S0)
    def _theta1():
        proc1_ref[...] = jnp.sum(jnp.exp(theta1_ref[...]), axis=2)


_SC_W = 16
_RPW = 512 // _SC_W
_ORW = 64 // _SC_W


def _sc_body(proc1_hbm, idx_hbm, obs1_hbm, l0_hbm, zeros_hbm, out_hbm,
             rows_v, idx_v, a_v, obs_v, pv_v, pall_v, l0_v, out_v,
             acc_sh, part_sh):
    w = lax.axis_index("s")

    pltpu.sync_copy(idx_hbm.at[pl.ds(w * _RPW, _RPW)], idx_v)
    pltpu.sync_copy(proc1_hbm.at[pl.ds(w * _RPW, _RPW)], rows_v)

    pltpu.sync_copy(zeros_hbm.at[pl.ds(w * _ORW, _ORW)],
                    acc_sh.at[pl.ds(w * _ORW, _ORW)])
    plsc.subcore_barrier()

    pltpu.sync_copy(rows_v, acc_sh.at[idx_v], add=True)
    plsc.subcore_barrier()

    @pl.when(w == 0)
    def _finish():
        acc = jnp.zeros((16,), jnp.float32)
        for blk in range(64 // _ORW):
            pltpu.sync_copy(acc_sh.at[pl.ds(blk * _ORW, _ORW)], a_v)
            pltpu.sync_copy(obs1_hbm.at[pl.ds(blk * _ORW, _ORW)], obs_v)
            for r in range(_ORW):
                for l in range(128 // 16):
                    d = obs_v[r, pl.ds(l * 16, 16)] - a_v[r, pl.ds(l * 16, 16)]
                    acc = acc + d * d
        lanes = jnp.arange(16, dtype=jnp.int32)
        for s in (1, 2, 4, 8):
            pv_v[...] = acc
            acc = acc + plsc.load_gather(pv_v, [(lanes + s) & 15])
        pltpu.sync_copy(l0_hbm, l0_v)
        l0vec = l0_v[0, pl.ds(0, 16)]
        out_v[...] = 0.5 * (l0vec + acc * (1.0 / (64.0 * 128.0)))
        pltpu.sync_copy(out_v, out_hbm)


@functools.partial(
    pl.kernel,
    out_type=jax.ShapeDtypeStruct((16,), jnp.float32),
    mesh=plsc.VectorSubcoreMesh(core_axis_name="c", subcore_axis_name="s",
                                num_cores=1),
    compiler_params=pltpu.CompilerParams(needs_layout_passes=False),
    scratch_types=[
        pltpu.VMEM((_RPW, 128), jnp.float32),
        pltpu.VMEM((_RPW,), jnp.int32),
        pltpu.VMEM((_ORW, 128), jnp.float32),
        pltpu.VMEM((_ORW, 128), jnp.float32),
        pltpu.VMEM((16,), jnp.float32),
        pltpu.VMEM((_SC_W, 16), jnp.float32),
        pltpu.VMEM((1, 128), jnp.float32),
        pltpu.VMEM((16,), jnp.float32),
        pltpu.VMEM_SHARED((64, 128), jnp.float32),
        pltpu.VMEM_SHARED((_SC_W, 16), jnp.float32),
    ],
)
def _sc_finish(proc1_hbm, idx_hbm, obs1_hbm, l0_hbm, zeros_hbm, out_hbm,
               *scratch):
    _sc_body(proc1_hbm, idx_hbm, obs1_hbm, l0_hbm, zeros_hbm, out_hbm,
             *scratch)


def kernel(theta0, theta1, obs0, obs1, mapping0, idx1):
    theta0_t = jnp.transpose(theta0, (0, 2, 1))
    proc1, l0 = pl.pallas_call(
        _tc_body,
        grid=(_STEPS,),
        in_specs=[
            pl.BlockSpec((_B0, 64, 256), lambda i: (jnp.minimum(i, _S0 - 1), 0, 0)),
            pl.BlockSpec((_B1, 128, 128), lambda i: (jnp.maximum(i - _S0, 0), 0, 0)),
            pl.BlockSpec((2048, 256), lambda i: (0, 0)),
            pl.BlockSpec((2048, 256), lambda i: (0, 0)),
        ],
        out_specs=[
            pl.BlockSpec((_B1, 128), lambda i: (jnp.maximum(i - _S0, 0), 0)),
            pl.BlockSpec((1, 128), lambda i: (0, 0)),
        ],
        out_shape=[
            jax.ShapeDtypeStruct((512, 128), jnp.float32),
            jax.ShapeDtypeStruct((1, 128), jnp.float32),
        ],
        scratch_shapes=[
            pltpu.VMEM((256, 256), jnp.float32),
        ],
        compiler_params=pltpu.CompilerParams(
            dimension_semantics=("arbitrary",),
        ),
    )(theta0_t, theta1, mapping0, obs0)
    zeros64 = jnp.zeros((64, 128), jnp.float32)
    out = _sc_finish(proc1, idx1.astype(jnp.int32), obs1, l0, zeros64)
    return out[0]


# speedup vs baseline: 1.1263x; 1.0031x over previous
import functools

import jax
import jax.numpy as jnp
from jax import lax
from jax.experimental import pallas as pl
from jax.experimental.pallas import tpu as pltpu
from jax.experimental.pallas import tpu_sc as plsc

_B0 = 128
_B1 = 128
_S0 = 2
_S1 = 4
_STEPS = _S0 + _S1


def _tc_body(theta0_ref, theta1_ref, map_ref, obs0_ref,
             proc1_ref, l0_ref, proc0_sc):
    i = pl.program_id(0)

    @pl.when(i < _S0)
    def _theta0():
        s0 = jnp.sum(jnp.exp(theta0_ref[...]), axis=1)
        proc0_sc[pl.ds(i * _B0, _B0), :] = s0

    @pl.when(i == _S0)
    def _loss0():
        p = jnp.dot(map_ref[...], proc0_sc[...],
                    preferred_element_type=jnp.float32)
        d0 = obs0_ref[...] - p
        l0 = jnp.sum(d0 * d0, keepdims=True) * (1.0 / (2048.0 * 256.0))
        l0_ref[...] = jnp.broadcast_to(l0.reshape(1, 1), (1, 128))

    @pl.when(i >= _S0)
    def _theta1():
        proc1_ref[...] = jnp.sum(jnp.exp(theta1_ref[...]), axis=2)


_SC_W = 16
_RPW = 512 // _SC_W
_ORW = 64 // _SC_W


def _sc_body(proc1_hbm, idx_hbm, obs1_hbm, l0_hbm, zeros_hbm, out_hbm,
             rows_v, idx_v, a_v, obs_v, pv_v, pall_v, l0_v, out_v,
             acc_sh, part_sh):
    w = lax.axis_index("s")

    pltpu.sync_copy(idx_hbm.at[pl.ds(w * _RPW, _RPW)], idx_v)
    pltpu.sync_copy(proc1_hbm.at[pl.ds(w * _RPW, _RPW)], rows_v)

    pltpu.sync_copy(zeros_hbm.at[pl.ds(w * _ORW, _ORW)],
                    acc_sh.at[pl.ds(w * _ORW, _ORW)])
    plsc.subcore_barrier()

    pltpu.sync_copy(rows_v, acc_sh.at[idx_v], add=True)
    plsc.subcore_barrier()

    @pl.when(w == 0)
    def _finish():
        acc = jnp.zeros((16,), jnp.float32)
        for blk in range(64 // _ORW):
            pltpu.sync_copy(acc_sh.at[pl.ds(blk * _ORW, _ORW)], a_v)
            pltpu.sync_copy(obs1_hbm.at[pl.ds(blk * _ORW, _ORW)], obs_v)
            for r in range(_ORW):
                for l in range(128 // 16):
                    d = obs_v[r, pl.ds(l * 16, 16)] - a_v[r, pl.ds(l * 16, 16)]
                    acc = acc + d * d
        lanes = jnp.arange(16, dtype=jnp.int32)
        for s in (1, 2, 4, 8):
            pv_v[...] = acc
            acc = acc + plsc.load_gather(pv_v, [(lanes + s) & 15])
        pltpu.sync_copy(l0_hbm, l0_v)
        l0vec = l0_v[0, pl.ds(0, 16)]
        out_v[...] = 0.5 * (l0vec + acc * (1.0 / (64.0 * 128.0)))
        pltpu.sync_copy(out_v, out_hbm)


@functools.partial(
    pl.kernel,
    out_type=jax.ShapeDtypeStruct((16,), jnp.float32),
    mesh=plsc.VectorSubcoreMesh(core_axis_name="c", subcore_axis_name="s",
                                num_cores=1),
    compiler_params=pltpu.CompilerParams(needs_layout_passes=False),
    scratch_types=[
        pltpu.VMEM((_RPW, 128), jnp.float32),
        pltpu.VMEM((_RPW,), jnp.int32),
        pltpu.VMEM((_ORW, 128), jnp.float32),
        pltpu.VMEM((_ORW, 128), jnp.float32),
        pltpu.VMEM((16,), jnp.float32),
        pltpu.VMEM((_SC_W, 16), jnp.float32),
        pltpu.VMEM((1, 128), jnp.float32),
        pltpu.VMEM((16,), jnp.float32),
        pltpu.VMEM_SHARED((64, 128), jnp.float32),
        pltpu.VMEM_SHARED((_SC_W, 16), jnp.float32),
    ],
)
def _sc_finish(proc1_hbm, idx_hbm, obs1_hbm, l0_hbm, zeros_hbm, out_hbm,
               *scratch):
    _sc_body(proc1_hbm, idx_hbm, obs1_hbm, l0_hbm, zeros_hbm, out_hbm,
             *scratch)


def kernel(theta0, theta1, obs0, obs1, mapping0, idx1):
    theta0_t = jnp.transpose(theta0, (0, 2, 1))
    proc1, l0 = pl.pallas_call(
        _tc_body,
        grid=(_STEPS,),
        in_specs=[
            pl.BlockSpec((_B0, 64, 256), lambda i: (jnp.minimum(i, _S0 - 1), 0, 0)),
            pl.BlockSpec((_B1, 128, 128), lambda i: (jnp.maximum(i - _S0, 0), 0, 0)),
            pl.BlockSpec((2048, 256), lambda i: (0, 0)),
            pl.BlockSpec((2048, 256), lambda i: (0, 0)),
        ],
        out_specs=[
            pl.BlockSpec((_B1, 128), lambda i: (jnp.maximum(i - _S0, 0), 0)),
            pl.BlockSpec((1, 128), lambda i: (0, 0)),
        ],
        out_shape=[
            jax.ShapeDtypeStruct((512, 128), jnp.float32),
            jax.ShapeDtypeStruct((1, 128), jnp.float32),
        ],
        scratch_shapes=[
            pltpu.VMEM((256, 256), jnp.float32),
        ],
        compiler_params=pltpu.CompilerParams(
            dimension_semantics=("arbitrary",),
        ),
    )(theta0_t, theta1, mapping0, obs0)
    zeros64 = jnp.zeros((64, 128), jnp.float32)
    out = _sc_finish(proc1, idx1.astype(jnp.int32), obs1, l0, zeros64)
    return out[0]


# speedup vs baseline: 1.1267x; 1.0004x over previous
import functools

import jax
import jax.numpy as jnp
from jax import lax
from jax.experimental import pallas as pl
from jax.experimental.pallas import tpu as pltpu
from jax.experimental.pallas import tpu_sc as plsc

_B0 = 128
_B1 = 128
_S0 = 2
_S1 = 4
_STEPS = _S0 + _S1


def _tc_body(theta0_ref, theta1_ref, map_ref, obs0_ref,
             proc1_ref, l0_ref, proc0_sc):
    i = pl.program_id(0)

    @pl.when(i < _S0)
    def _theta0():
        s0 = jnp.sum(jnp.exp(theta0_ref[...]), axis=1)
        proc0_sc[pl.ds(i * _B0, _B0), :] = s0

    @pl.when(i == _S0)
    def _loss0():
        p = jnp.dot(map_ref[...], proc0_sc[...],
                    preferred_element_type=jnp.float32)
        d0 = obs0_ref[...] - p
        l0 = jnp.sum(d0 * d0, keepdims=True) * (1.0 / (2048.0 * 256.0))
        l0_ref[...] = jnp.broadcast_to(l0.reshape(1, 1), (1, 128))

    @pl.when(i >= _S0)
    def _theta1():
        proc1_ref[...] = jnp.sum(jnp.exp(theta1_ref[...]), axis=2)


_SC_W = 16
_RPW = 512 // _SC_W
_ORW = 64 // _SC_W


def _sc_body(proc1_hbm, idx_hbm, obs1_hbm, l0_hbm, zeros_hbm, out_hbm,
             rows_v, idx_v, a_v, obs_v, pv_v, l0_v, out_v, acc_sh):
    w = lax.axis_index("s")

    pltpu.sync_copy(idx_hbm.at[pl.ds(w * _RPW, _RPW)], idx_v)
    pltpu.sync_copy(proc1_hbm.at[pl.ds(w * _RPW, _RPW)], rows_v)

    pltpu.sync_copy(zeros_hbm.at[pl.ds(w * _ORW, _ORW)],
                    acc_sh.at[pl.ds(w * _ORW, _ORW)])
    plsc.subcore_barrier()

    pltpu.sync_copy(rows_v, acc_sh.at[idx_v], add=True)
    plsc.subcore_barrier()

    @pl.when(w == 0)
    def _finish():
        acc = jnp.zeros((16,), jnp.float32)
        for blk in range(64 // _ORW):
            pltpu.sync_copy(acc_sh.at[pl.ds(blk * _ORW, _ORW)], a_v)
            pltpu.sync_copy(obs1_hbm.at[pl.ds(blk * _ORW, _ORW)], obs_v)
            for r in range(_ORW):
                for l in range(128 // 16):
                    d = obs_v[r, pl.ds(l * 16, 16)] - a_v[r, pl.ds(l * 16, 16)]
                    acc = acc + d * d
        lanes = jnp.arange(16, dtype=jnp.int32)
        for s in (1, 2, 4, 8):
            pv_v[...] = acc
            acc = acc + plsc.load_gather(pv_v, [(lanes + s) & 15])
        pltpu.sync_copy(l0_hbm, l0_v)
        l0vec = l0_v[0, pl.ds(0, 16)]
        out_v[...] = 0.5 * (l0vec + acc * (1.0 / (64.0 * 128.0)))
        pltpu.sync_copy(out_v, out_hbm)


@functools.partial(
    pl.kernel,
    out_type=jax.ShapeDtypeStruct((16,), jnp.float32),
    mesh=plsc.VectorSubcoreMesh(core_axis_name="c", subcore_axis_name="s",
                                num_cores=1),
    compiler_params=pltpu.CompilerParams(needs_layout_passes=False),
    scratch_types=[
        pltpu.VMEM((_RPW, 128), jnp.float32),
        pltpu.VMEM((_RPW,), jnp.int32),
        pltpu.VMEM((_ORW, 128), jnp.float32),
        pltpu.VMEM((_ORW, 128), jnp.float32),
        pltpu.VMEM((16,), jnp.float32),
        pltpu.VMEM((1, 128), jnp.float32),
        pltpu.VMEM((16,), jnp.float32),
        pltpu.VMEM_SHARED((64, 128), jnp.float32),
    ],
)
def _sc_finish(proc1_hbm, idx_hbm, obs1_hbm, l0_hbm, zeros_hbm, out_hbm,
               *scratch):
    _sc_body(proc1_hbm, idx_hbm, obs1_hbm, l0_hbm, zeros_hbm, out_hbm,
             *scratch)


def kernel(theta0, theta1, obs0, obs1, mapping0, idx1):
    theta0_t = jnp.transpose(theta0, (0, 2, 1))
    proc1, l0 = pl.pallas_call(
        _tc_body,
        grid=(_STEPS,),
        in_specs=[
            pl.BlockSpec((_B0, 64, 256), lambda i: (jnp.minimum(i, _S0 - 1), 0, 0)),
            pl.BlockSpec((_B1, 128, 128), lambda i: (jnp.maximum(i - _S0, 0), 0, 0)),
            pl.BlockSpec((2048, 256), lambda i: (0, 0)),
            pl.BlockSpec((2048, 256), lambda i: (0, 0)),
        ],
        out_specs=[
            pl.BlockSpec((_B1, 128), lambda i: (jnp.maximum(i - _S0, 0), 0)),
            pl.BlockSpec((1, 128), lambda i: (0, 0)),
        ],
        out_shape=[
            jax.ShapeDtypeStruct((512, 128), jnp.float32),
            jax.ShapeDtypeStruct((1, 128), jnp.float32),
        ],
        scratch_shapes=[
            pltpu.VMEM((256, 256), jnp.float32),
        ],
        compiler_params=pltpu.CompilerParams(
            dimension_semantics=("arbitrary",),
        ),
    )(theta0_t, theta1, mapping0, obs0)
    zeros64 = jnp.zeros((64, 128), jnp.float32)
    out = _sc_finish(proc1, idx1.astype(jnp.int32), obs1, l0, zeros64)
    return out[0]
